# baseline (speedup 1.0000x reference)
import functools

import jax
import jax.numpy as jnp
from jax import lax
from jax.experimental import pallas as pl
from jax.experimental.pallas import tpu as pltpu
from jax.experimental.pallas import tpu_sc as plsc

VOCAB = 100000
EMBED_DIM = 32
BATCH = 1024
CTX = 20

LANES = 16
VB = 2048
IDX_CHUNK = 128


def _pool_kernel(nc, ns):
    nw = nc * ns
    b_per_w = BATCH // nw
    idx_per_w = b_per_w * CTX
    n_chunks = idx_per_w // IDX_CHUNK

    mesh = plsc.VectorSubcoreMesh(core_axis_name="c", subcore_axis_name="s")

    @functools.partial(
        pl.kernel,
        mesh=mesh,
        compiler_params=pltpu.CompilerParams(use_tc_tiling_on_sc=False),
        out_type=jax.ShapeDtypeStruct((BATCH, EMBED_DIM), jnp.float32),
        scratch_types=[
            pltpu.VMEM((idx_per_w,), jnp.int32),
            pltpu.VMEM((idx_per_w, EMBED_DIM), jnp.float32),
            pltpu.VMEM((b_per_w, EMBED_DIM), jnp.float32),
            pltpu.SemaphoreType.DMA,
        ],
    )
    def pool(idx_hbm, table_hbm, out_hbm, idx_v, rows_v, pool_v, sem):
        wid = lax.axis_index("s") * nc + lax.axis_index("c")
        pltpu.sync_copy(idx_hbm.at[pl.ds(wid * idx_per_w, idx_per_w)], idx_v)
        copies = []
        for j in range(n_chunks):
            copies.append(
                pltpu.async_copy(
                    table_hbm.at[idx_v.at[pl.ds(j * IDX_CHUNK, IDX_CHUNK)]],
                    rows_v.at[pl.ds(j * IDX_CHUNK, IDX_CHUNK)],
                    sem,
                )
            )
        for c in copies:
            c.wait()

        def body(i, _):
            acc0 = jnp.zeros((LANES,), jnp.float32)
            acc1 = jnp.zeros((LANES,), jnp.float32)
            for j in range(CTX):
                r = i * CTX + j
                acc0 = acc0 + rows_v[r, pl.ds(0, LANES)]
                acc1 = acc1 + rows_v[r, pl.ds(LANES, LANES)]
            pool_v[i, pl.ds(0, LANES)] = acc0 * (1.0 / CTX)
            pool_v[i, pl.ds(LANES, LANES)] = acc1 * (1.0 / CTX)
            return 0

        lax.fori_loop(0, b_per_w, body, 0)
        pltpu.sync_copy(pool_v, out_hbm.at[pl.ds(wid * b_per_w, b_per_w)])

    return pool


def _mm_body(pt_ref, w_ref, b_ref, o_ref):
    o = lax.dot_general(
        w_ref[...], pt_ref[...],
        (((0,), (0,)), ((), ())),
        preferred_element_type=jnp.float32,
    )
    o_ref[...] = o + jnp.transpose(b_ref[...])


def _matmul_t(pooled_t, W, b2):
    nvb = pl.cdiv(VOCAB, VB)
    return pl.pallas_call(
        _mm_body,
        grid=(nvb,),
        in_specs=[
            pl.BlockSpec((EMBED_DIM, BATCH), lambda i: (0, 0)),
            pl.BlockSpec((EMBED_DIM, VB), lambda i: (0, i)),
            pl.BlockSpec((1, VB), lambda i: (0, i)),
        ],
        out_specs=pl.BlockSpec((VB, BATCH), lambda i: (i, 0)),
        out_shape=jax.ShapeDtypeStruct((VOCAB, BATCH), jnp.float32),
    )(pooled_t, W, b2)


def kernel(inputs, table, W, b):
    info = plsc.get_sparse_core_info()
    nc, ns = info.num_cores, info.num_subcores
    idx = inputs.astype(jnp.int32).reshape(-1)
    pooled = _pool_kernel(nc, ns)(idx, table)
    out_t = _matmul_t(pooled.T, W, b.reshape(1, VOCAB))
    return out_t.T


# speedup vs baseline: 1.0021x; 1.0021x over previous
import functools

import jax
import jax.numpy as jnp
from jax import lax
from jax.experimental import pallas as pl
from jax.experimental.pallas import tpu as pltpu
from jax.experimental.pallas import tpu_sc as plsc

VOCAB = 100000
EMBED_DIM = 32
BATCH = 1024
CTX = 20

LANES = 16
VB = 2048
IDX_CHUNK = 128


def _pool_kernel(nc, ns):
    nw = nc * ns
    b_per_w = BATCH // nw
    idx_per_w = b_per_w * CTX
    n_chunks = idx_per_w // IDX_CHUNK

    mesh = plsc.VectorSubcoreMesh(core_axis_name="c", subcore_axis_name="s")

    @functools.partial(
        pl.kernel,
        mesh=mesh,
        compiler_params=pltpu.CompilerParams(use_tc_tiling_on_sc=True),
        out_type=jax.ShapeDtypeStruct((BATCH, EMBED_DIM), jnp.float32),
        scratch_types=[
            pltpu.VMEM((idx_per_w,), jnp.int32),
            pltpu.VMEM((idx_per_w, 128), jnp.float32),
            pltpu.VMEM((b_per_w, EMBED_DIM), jnp.float32),
            pltpu.SemaphoreType.DMA,
        ],
    )
    def pool(idx_hbm, table_hbm, out_hbm, idx_v, rows_v, pool_v, sem):
        wid = lax.axis_index("s") * nc + lax.axis_index("c")
        pltpu.sync_copy(idx_hbm.at[pl.ds(wid * idx_per_w, idx_per_w)], idx_v)
        copies = []
        for j in range(n_chunks):
            copies.append(
                pltpu.async_copy(
                    table_hbm.at[idx_v.at[pl.ds(j * IDX_CHUNK, IDX_CHUNK)]],
                    rows_v.at[pl.ds(j * IDX_CHUNK, IDX_CHUNK)],
                    sem,
                )
            )
        for c in copies:
            c.wait()

        def body(i, _):
            acc0 = jnp.zeros((LANES,), jnp.float32)
            acc1 = jnp.zeros((LANES,), jnp.float32)
            for j in range(CTX):
                r = i * CTX + j
                acc0 = acc0 + rows_v[r, pl.ds(0, LANES)]
                acc1 = acc1 + rows_v[r, pl.ds(LANES, LANES)]
            pool_v[i, pl.ds(0, LANES)] = acc0 * (1.0 / CTX)
            pool_v[i, pl.ds(LANES, LANES)] = acc1 * (1.0 / CTX)
            return 0

        lax.fori_loop(0, b_per_w, body, 0)
        pltpu.sync_copy(pool_v, out_hbm.at[pl.ds(wid * b_per_w, b_per_w)])

    return pool


def _mm_body(pt_ref, w_ref, b_ref, o_ref):
    o = lax.dot_general(
        w_ref[...], pt_ref[...],
        (((0,), (0,)), ((), ())),
        preferred_element_type=jnp.float32,
    )
    o_ref[...] = o + jnp.transpose(b_ref[...])


def _matmul_t(pooled_t, W, b2):
    nvb = pl.cdiv(VOCAB, VB)
    return pl.pallas_call(
        _mm_body,
        grid=(nvb,),
        in_specs=[
            pl.BlockSpec((EMBED_DIM, BATCH), lambda i: (0, 0)),
            pl.BlockSpec((EMBED_DIM, VB), lambda i: (0, i)),
            pl.BlockSpec((1, VB), lambda i: (0, i)),
        ],
        out_specs=pl.BlockSpec((VB, BATCH), lambda i: (i, 0)),
        out_shape=jax.ShapeDtypeStruct((VOCAB, BATCH), jnp.float32),
    )(pooled_t, W, b2)


def kernel(inputs, table, W, b):
    info = plsc.get_sparse_core_info()
    nc, ns = info.num_cores, info.num_subcores
    idx = inputs.astype(jnp.int32).reshape(-1)
    table128 = jnp.pad(table, ((0, 0), (0, 128 - EMBED_DIM)))
    pooled = _pool_kernel(nc, ns)(idx, table128)
    out_t = _matmul_t(pooled.T, W, b.reshape(1, VOCAB))
    return out_t.T
